# baseline (speedup 1.0000x reference)
import functools

import jax
import jax.numpy as jnp
from jax import lax
from jax.experimental import pallas as pl
from jax.experimental.pallas import tpu as pltpu
from jax.experimental.pallas import tpu_sc as plsc

N_NODES = 100000
FEAT = 128
NW = 32
CH = 16
ROWS = 7 * CH
CPW = 195
UNROLL = 4

_mesh = plsc.VectorSubcoreMesh(core_axis_name="c", subcore_axis_name="s")


@functools.partial(
    pl.kernel,
    mesh=_mesh,
    compiler_params=pltpu.CompilerParams(needs_layout_passes=False),
    out_type=[
        jax.ShapeDtypeStruct((N_NODES, FEAT), jnp.float32),
        jax.ShapeDtypeStruct((N_NODES, FEAT), jnp.int32),
    ],
    scratch_types=[
        pltpu.VMEM((ROWS,), jnp.int32),
        pltpu.VMEM((ROWS, FEAT), jnp.float32),
        pltpu.VMEM((CH, FEAT), jnp.float32),
        pltpu.VMEM((CH, FEAT), jnp.int32),
        pltpu.SemaphoreType.DMA,
    ],
)
def _sc_pool(x_hbm, no_hbm, vals_hbm, idxs_hbm, idx_v, rows_v, vout_v, iout_v, sem):
    wid = lax.axis_index("s") * 2 + lax.axis_index("c")
    node0 = CH * CPW * wid + CH * jnp.minimum(wid, 10)
    n_chunks = jnp.where(wid < 10, CPW + 1, CPW)

    iota = lax.iota(jnp.int32, 16)
    iota7 = iota * 7
    kf = [jnp.full((16,), float(k), jnp.float32) for k in range(7)]

    def chunk_body(g, _):
        node_base = node0 + g * CH
        pltpu.sync_copy(no_hbm.at[pl.ds(node_base * 7, ROWS)], idx_v)
        pltpu.async_copy(x_hbm.at[idx_v], rows_v, sem).wait()

        for f in range(FEAT):
            p0 = f * 7
            bval = None
            bidx = None
            for k in range(7):
                p = p0 + k
                col = jnp.full((16,), p & 127, jnp.int32)
                gv = plsc.load_gather(rows_v, [iota7 + (p >> 7), col])
                if k == 0:
                    bval = gv
                    bidx = kf[0]
                else:
                    m = gv > bval
                    bval = jnp.maximum(bval, gv)
                    bidx = jnp.where(m, kf[k], bidx)
            colf = jnp.full((16,), f, jnp.int32)
            plsc.store_scatter(vout_v, [iota, colf], bval)
            plsc.store_scatter(iout_v, [iota, colf], bidx.astype(jnp.int32))
        pltpu.sync_copy(vout_v, vals_hbm.at[pl.ds(node_base, CH)])
        pltpu.sync_copy(iout_v, idxs_hbm.at[pl.ds(node_base, CH)])
        return 0

    lax.fori_loop(0, n_chunks, chunk_body, 0)


def kernel(x, neigh_orders):
    no32 = neigh_orders.astype(jnp.int32)
    vals, idxs = _sc_pool(x, no32)
    return (vals, idxs)


# speedup vs baseline: 1.0773x; 1.0773x over previous
import functools

import jax
import jax.numpy as jnp
from jax import lax
from jax.experimental import pallas as pl
from jax.experimental.pallas import tpu as pltpu
from jax.experimental.pallas import tpu_sc as plsc

N_NODES = 100000
FEAT = 128
NW = 32
CH = 16
ROWS = 7 * CH
HROWS = ROWS // 2
CPW_LO = 194
IDX_CAP = 200 * ROWS
NO_PAD = 7 * 96896 + IDX_CAP

_mesh = plsc.VectorSubcoreMesh(core_axis_name="c", subcore_axis_name="s")


@functools.partial(
    pl.kernel,
    mesh=_mesh,
    compiler_params=pltpu.CompilerParams(needs_layout_passes=False),
    out_type=[
        jax.ShapeDtypeStruct((N_NODES, FEAT), jnp.float32),
        jax.ShapeDtypeStruct((N_NODES, FEAT), jnp.int32),
    ],
    scratch_types=[
        pltpu.VMEM((IDX_CAP,), jnp.int32),
        pltpu.VMEM((ROWS, FEAT), jnp.float32),
        pltpu.VMEM((ROWS, FEAT), jnp.float32),
        pltpu.VMEM((CH, FEAT), jnp.float32),
        pltpu.VMEM((CH, FEAT), jnp.float32),
        pltpu.VMEM((CH, FEAT), jnp.int32),
        pltpu.VMEM((CH, FEAT), jnp.int32),
        pltpu.SemaphoreType.DMA,
        pltpu.SemaphoreType.DMA,
        pltpu.SemaphoreType.DMA,
        pltpu.SemaphoreType.DMA,
    ],
)
def _sc_pool(x_hbm, no_hbm, vals_hbm, idxs_hbm,
             idx_all, rows0, rows1, vout0, vout1, iout0, iout1,
             sem_g0, sem_g1, sem_o0, sem_o1):
    wid = lax.axis_index("s") * 2 + lax.axis_index("c")
    node0 = CH * CPW_LO * wid + 2 * CH * jnp.minimum(wid, 21)
    n_pairs = jnp.where(wid < 21, (CPW_LO + 2) // 2, CPW_LO // 2)

    iota = lax.iota(jnp.int32, 16)
    row_vecs = [iota * 7 + r for r in range(7)]
    kf = [jnp.full((16,), float(k), jnp.float32) for k in range(7)]

    pltpu.sync_copy(no_hbm.at[pl.ds(node0 * 7, IDX_CAP)], idx_all)

    def gather(g, rows_ref, sem):
        base = g * ROWS
        pltpu.async_copy(
            x_hbm.at[idx_all.at[pl.ds(base, HROWS)]],
            rows_ref.at[pl.ds(0, HROWS)], sem)
        pltpu.async_copy(
            x_hbm.at[idx_all.at[pl.ds(base + HROWS, HROWS)]],
            rows_ref.at[pl.ds(HROWS, HROWS)], sem)

    def wait_gather(rows_ref, sem):
        for h in range(2):
            pltpu.make_async_copy(
                x_hbm.at[idx_all.at[pl.ds(0, HROWS)]],
                rows_ref.at[pl.ds(h * HROWS, HROWS)], sem).wait()

    def put_out(g, vout, iout, sem):
        node_base = node0 + g * CH
        pltpu.async_copy(vout, vals_hbm.at[pl.ds(node_base, CH)], sem)
        pltpu.async_copy(iout, idxs_hbm.at[pl.ds(node_base, CH)], sem)

    def wait_out(vout, iout, sem):
        pltpu.make_async_copy(vout, vals_hbm.at[pl.ds(0, CH)], sem).wait()
        pltpu.make_async_copy(iout, idxs_hbm.at[pl.ds(0, CH)], sem).wait()

    def compute(rows_ref, vout, iout):
        for f in range(FEAT):
            p0 = f * 7
            bval = None
            bidx = None
            for k in range(7):
                p = p0 + k
                col = jnp.full((16,), p & 127, jnp.int32)
                gv = plsc.load_gather(rows_ref, [row_vecs[p >> 7], col])
                if k == 0:
                    bval = gv
                    bidx = kf[0]
                else:
                    m = gv > bval
                    bval = jnp.maximum(bval, gv)
                    bidx = jnp.where(m, kf[k], bidx)
            colf = jnp.full((16,), f, jnp.int32)
            plsc.store_scatter(vout, [iota, colf], bval)
            plsc.store_scatter(iout, [iota, colf], bidx.astype(jnp.int32))

    gather(0, rows0, sem_g0)

    def pair_body(m, _):
        g0 = 2 * m
        gather(g0 + 1, rows1, sem_g1)
        wait_gather(rows0, sem_g0)

        @pl.when(m > 0)
        def _():
            wait_out(vout0, iout0, sem_o0)

        compute(rows0, vout0, iout0)
        put_out(g0, vout0, iout0, sem_o0)
        gather(g0 + 2, rows0, sem_g0)

        wait_gather(rows1, sem_g1)

        @pl.when(m > 0)
        def _():
            wait_out(vout1, iout1, sem_o1)

        compute(rows1, vout1, iout1)
        put_out(g0 + 1, vout1, iout1, sem_o1)
        return 0

    lax.fori_loop(0, n_pairs, pair_body, 0)

    wait_gather(rows0, sem_g0)
    wait_out(vout0, iout0, sem_o0)
    wait_out(vout1, iout1, sem_o1)


def kernel(x, neigh_orders):
    no32 = neigh_orders.astype(jnp.int32)
    no32 = jnp.concatenate(
        [no32, jnp.zeros((NO_PAD - no32.shape[0],), jnp.int32)])
    vals, idxs = _sc_pool(x, no32)
    return (vals, idxs)


# speedup vs baseline: 4.9979x; 4.6391x over previous
import functools

import jax
import jax.numpy as jnp
from jax import lax
from jax.experimental import pallas as pl
from jax.experimental.pallas import tpu as pltpu
from jax.experimental.pallas import tpu_sc as plsc

N_NODES = 100000
FEAT = 128
NW = 32
CH = 16
ROWS = 7 * CH
HROWS = ROWS // 2
CPW_LO = 194
IDX_CAP = 200 * ROWS
NO_PAD = 7 * 96896 + IDX_CAP

_mesh = plsc.VectorSubcoreMesh(core_axis_name="c", subcore_axis_name="s")


@functools.partial(
    pl.kernel,
    mesh=_mesh,
    compiler_params=pltpu.CompilerParams(needs_layout_passes=False),
    out_type=[
        jax.ShapeDtypeStruct((N_NODES, FEAT), jnp.float32),
        jax.ShapeDtypeStruct((N_NODES, FEAT), jnp.int32),
    ],
    scratch_types=[
        pltpu.VMEM((IDX_CAP,), jnp.int32),
        pltpu.VMEM((ROWS, FEAT), jnp.float32),
        pltpu.VMEM((ROWS, FEAT), jnp.float32),
        pltpu.VMEM((CH, FEAT), jnp.float32),
        pltpu.VMEM((CH, FEAT), jnp.float32),
        pltpu.VMEM((CH, FEAT), jnp.int32),
        pltpu.VMEM((CH, FEAT), jnp.int32),
        pltpu.SemaphoreType.DMA,
        pltpu.SemaphoreType.DMA,
        pltpu.SemaphoreType.DMA,
        pltpu.SemaphoreType.DMA,
    ],
)
def _sc_pool(x_hbm, no_hbm, vals_hbm, idxs_hbm,
             idx_all, rows0, rows1, vout0, vout1, iout0, iout1,
             sem_g0, sem_g1, sem_o0, sem_o1):
    wid = lax.axis_index("s") * 2 + lax.axis_index("c")
    node0 = CH * CPW_LO * wid + 2 * CH * jnp.minimum(wid, 21)
    n_pairs = jnp.where(wid < 21, (CPW_LO + 2) // 2, CPW_LO // 2)

    iota = lax.iota(jnp.int32, 16)
    iota7 = iota * 7
    p_vecs = [iota7 + (112 * v) if v else iota7 for v in range(8)]
    kf = [jnp.full((16,), float(k), jnp.float32) for k in range(7)]

    pltpu.sync_copy(no_hbm.at[pl.ds(node0 * 7, IDX_CAP)], idx_all)

    def gather(g, rows_ref, sem):
        base = g * ROWS
        pltpu.async_copy(
            x_hbm.at[idx_all.at[pl.ds(base, HROWS)]],
            rows_ref.at[pl.ds(0, HROWS)], sem)
        pltpu.async_copy(
            x_hbm.at[idx_all.at[pl.ds(base + HROWS, HROWS)]],
            rows_ref.at[pl.ds(HROWS, HROWS)], sem)

    def wait_gather(rows_ref, sem):
        for h in range(2):
            pltpu.make_async_copy(
                x_hbm.at[idx_all.at[pl.ds(0, HROWS)]],
                rows_ref.at[pl.ds(h * HROWS, HROWS)], sem).wait()

    def put_out(g, vout, iout, sem):
        node_base = node0 + g * CH
        pltpu.async_copy(vout, vals_hbm.at[pl.ds(node_base, CH)], sem)
        pltpu.async_copy(iout, idxs_hbm.at[pl.ds(node_base, CH)], sem)

    def wait_out(vout, iout, sem):
        pltpu.make_async_copy(vout, vals_hbm.at[pl.ds(0, CH)], sem).wait()
        pltpu.make_async_copy(iout, idxs_hbm.at[pl.ds(0, CH)], sem).wait()

    def compute(rows_ref, vout, iout):
        def node_body(b, _):
            row_off = b * 7
            for v in range(8):
                bval = None
                bidx = None
                for k in range(7):
                    pk = p_vecs[v] + k if k else p_vecs[v]
                    row = (pk >> 7) + row_off
                    col = pk & 127
                    gv = plsc.load_gather(rows_ref, [row, col])
                    if k == 0:
                        bval = gv
                        bidx = kf[0]
                    else:
                        m = gv > bval
                        bval = jnp.maximum(bval, gv)
                        bidx = jnp.where(m, kf[k], bidx)
                vout[b, pl.ds(16 * v, 16)] = bval
                iout[b, pl.ds(16 * v, 16)] = bidx.astype(jnp.int32)
            return 0

        lax.fori_loop(0, CH, node_body, 0)

    gather(0, rows0, sem_g0)

    def pair_body(m, _):
        g0 = 2 * m
        gather(g0 + 1, rows1, sem_g1)
        wait_gather(rows0, sem_g0)

        @pl.when(m > 0)
        def _():
            wait_out(vout0, iout0, sem_o0)

        compute(rows0, vout0, iout0)
        put_out(g0, vout0, iout0, sem_o0)
        gather(g0 + 2, rows0, sem_g0)

        wait_gather(rows1, sem_g1)

        @pl.when(m > 0)
        def _():
            wait_out(vout1, iout1, sem_o1)

        compute(rows1, vout1, iout1)
        put_out(g0 + 1, vout1, iout1, sem_o1)
        return 0

    lax.fori_loop(0, n_pairs, pair_body, 0)

    wait_gather(rows0, sem_g0)
    wait_out(vout0, iout0, sem_o0)
    wait_out(vout1, iout1, sem_o1)


def kernel(x, neigh_orders):
    no32 = neigh_orders.astype(jnp.int32)
    no32 = jnp.concatenate(
        [no32, jnp.zeros((NO_PAD - no32.shape[0],), jnp.int32)])
    vals, idxs = _sc_pool(x, no32)
    return (vals, idxs)
